# baseline (speedup 1.0000x reference)
import functools

import jax
import jax.numpy as jnp
from jax import lax
from jax.experimental import pallas as pl
from jax.experimental.pallas import tpu as pltpu
from jax.experimental.pallas import tpu_sc as plsc

M, L, H, W = 128, 1024, 256, 256
NNZ = 8_000_000
B = 4
ROWS = M * L
COLS = H * W

WSZ = 3200
NWIN = NNZ // WSZ
NCORE = 2
NSUB = 16
WIN_PER_CORE = NWIN // NCORE
SUB_Q = WIN_PER_CORE // NSUB
SUB_R = WIN_PER_CORE - SUB_Q * NSUB

PSLICE = COLS // NSUB
YSLICE = ROWS // NSUB
ZB = 4096


def _sc_body(vals_hbm, rows_hbm, cols_hbm, p01_hbm, p23_hbm, out_hbm,
             p01_sh, p23_sh,
             y0_sh, y1_sh, y2_sh, y3_sh,
             vals_a, rows_a, cols_a, ppa01, ppa23, ua0, ua1, ua2, ua3,
             vals_b, rows_b, cols_b, ppb01, ppb23, ub0, ub1, ub2, ub3,
             zbuf, sem_lin, sem_gat, sem_sca):
  c = lax.axis_index("c")
  s = lax.axis_index("s")
  y_shs = [y0_sh, y1_sh, y2_sh, y3_sh]

  def _zero_zbuf(j, _):
    zbuf[pl.ds(j * 16, 16)] = jnp.zeros((16,), jnp.float32)
    return _
  lax.fori_loop(0, ZB // 16, _zero_zbuf, None)

  for b in range(B):
    for k in range(YSLICE // ZB):
      pltpu.sync_copy(zbuf, y_shs[b].at[pl.ds(s * YSLICE + k * ZB, ZB)])
  pltpu.sync_copy(p01_hbm.at[pl.ds(s * PSLICE, PSLICE)],
                  p01_sh.at[pl.ds(s * PSLICE, PSLICE)])
  pltpu.sync_copy(p23_hbm.at[pl.ds(s * PSLICE, PSLICE)],
                  p23_sh.at[pl.ds(s * PSLICE, PSLICE)])
  plsc.subcore_barrier()

  start = c * WIN_PER_CORE + s * SUB_Q + jnp.minimum(s, SUB_R)
  nwin = SUB_Q + jnp.where(s < SUB_R, 1, 0)
  SETS = (
      (vals_a, rows_a, cols_a, (ppa01, ppa23), (ua0, ua1, ua2, ua3)),
      (vals_b, rows_b, cols_b, (ppb01, ppb23), (ub0, ub1, ub2, ub3)),
  )

  def _issue_linear(w, t):
    vals_v, rows_v, cols_v, _, _ = SETS[t]
    base = (start + w) * WSZ
    pltpu.async_copy(vals_hbm.at[pl.ds(base, WSZ)], vals_v, sem_lin)
    pltpu.async_copy(rows_hbm.at[pl.ds(base, WSZ)], rows_v, sem_lin)
    pltpu.async_copy(cols_hbm.at[pl.ds(base, WSZ)], cols_v, sem_lin)

  def _drain_linear(t):
    vals_v, rows_v, cols_v, _, _ = SETS[t]
    pltpu.make_async_copy(vals_hbm.at[pl.ds(0, WSZ)], vals_v, sem_lin).wait()
    pltpu.make_async_copy(rows_hbm.at[pl.ds(0, WSZ)], rows_v, sem_lin).wait()
    pltpu.make_async_copy(cols_hbm.at[pl.ds(0, WSZ)], cols_v, sem_lin).wait()

  def _issue_gathers(t):
    _, _, cols_v, pps, _ = SETS[t]
    pltpu.async_copy(p01_sh.at[cols_v], pps[0], sem_gat)
    pltpu.async_copy(p23_sh.at[cols_v], pps[1], sem_gat)

  def _drain_gathers(t):
    _, _, cols_v, pps, _ = SETS[t]
    pltpu.make_async_copy(p01_sh.at[cols_v], pps[0], sem_gat).wait()
    pltpu.make_async_copy(p23_sh.at[cols_v], pps[1], sem_gat).wait()

  def _issue_scatters(t):
    _, rows_v, _, _, uds = SETS[t]
    for b in range(B):
      pltpu.async_copy(uds[b], y_shs[b].at[rows_v], sem_sca, add=True)

  def _drain_scatters(t):
    _, rows_v, _, _, uds = SETS[t]
    for b in range(B):
      pltpu.make_async_copy(uds[b], y_shs[b].at[rows_v], sem_sca).wait()

  def _mul(t):
    vals_v, _, _, pps, uds = SETS[t]
    mask_hi = jnp.int32(-65536)

    def body(j, _):
      sl = pl.ds(j * 16, 16)
      v = vals_v[sl]
      w01 = pps[0][sl]
      w23 = pps[1][sl]
      bc = jax.lax.bitcast_convert_type
      uds[0][sl] = bc(w01 & mask_hi, jnp.float32) * v
      uds[1][sl] = bc(w01 << 16, jnp.float32) * v
      uds[2][sl] = bc(w23 & mask_hi, jnp.float32) * v
      uds[3][sl] = bc(w23 << 16, jnp.float32) * v
      return _
    lax.fori_loop(0, WSZ // 16, body, None)

  npair = nwin // 2
  tail = nwin - 2 * npair

  _issue_linear(0, 0)
  _drain_linear(0)
  _issue_gathers(0)

  def _pair(q, _):
    wb = 2 * q + 1
    _drain_gathers(0)

    @pl.when(q >= 1)
    def _():
      _drain_scatters(1)
    _issue_linear(wb, 1)
    _mul(0)
    _issue_scatters(0)
    _drain_linear(1)
    _issue_gathers(1)
    _drain_gathers(1)
    _drain_scatters(0)

    @pl.when(wb + 1 < nwin)
    def _():
      _issue_linear(wb + 1, 0)
    _mul(1)
    _issue_scatters(1)

    @pl.when(wb + 1 < nwin)
    def _():
      _drain_linear(0)
      _issue_gathers(0)
    return _

  lax.fori_loop(0, npair, _pair, None)

  @pl.when(tail == 1)
  def _():
    _drain_gathers(0)
    _drain_scatters(1)
    _mul(0)
    _issue_scatters(0)
    _drain_scatters(0)

  @pl.when(tail == 0)
  def _():
    _drain_scatters(1)
  plsc.subcore_barrier()

  for b in range(B):
    pltpu.sync_copy(y_shs[b].at[pl.ds(s * YSLICE, YSLICE)],
                    out_hbm.at[c, b, pl.ds(s * YSLICE, YSLICE)])


_sc_spmv = functools.partial(
    pl.kernel,
    out_type=jax.ShapeDtypeStruct((NCORE, B, ROWS), jnp.float32),
    mesh=plsc.VectorSubcoreMesh(core_axis_name="c", subcore_axis_name="s"),
    scratch_types=[
        pltpu.VMEM_SHARED((COLS,), jnp.int32),
        pltpu.VMEM_SHARED((COLS,), jnp.int32),
        pltpu.VMEM_SHARED((ROWS,), jnp.float32),
        pltpu.VMEM_SHARED((ROWS,), jnp.float32),
        pltpu.VMEM_SHARED((ROWS,), jnp.float32),
        pltpu.VMEM_SHARED((ROWS,), jnp.float32),
        pltpu.VMEM((WSZ,), jnp.float32),
        pltpu.VMEM((WSZ,), jnp.int32),
        pltpu.VMEM((WSZ,), jnp.int32),
        pltpu.VMEM((WSZ,), jnp.int32),
        pltpu.VMEM((WSZ,), jnp.int32),
        pltpu.VMEM((WSZ,), jnp.float32),
        pltpu.VMEM((WSZ,), jnp.float32),
        pltpu.VMEM((WSZ,), jnp.float32),
        pltpu.VMEM((WSZ,), jnp.float32),
        pltpu.VMEM((WSZ,), jnp.float32),
        pltpu.VMEM((WSZ,), jnp.int32),
        pltpu.VMEM((WSZ,), jnp.int32),
        pltpu.VMEM((WSZ,), jnp.int32),
        pltpu.VMEM((WSZ,), jnp.int32),
        pltpu.VMEM((WSZ,), jnp.float32),
        pltpu.VMEM((WSZ,), jnp.float32),
        pltpu.VMEM((WSZ,), jnp.float32),
        pltpu.VMEM((WSZ,), jnp.float32),
        pltpu.VMEM((ZB,), jnp.float32),
        pltpu.SemaphoreType.DMA,
        pltpu.SemaphoreType.DMA,
        pltpu.SemaphoreType.DMA,
    ],
)(_sc_body)


def _norm_body(ypart_ref, out_ref):
  y = ypart_ref[0] + ypart_ref[1]
  m = jnp.max(y, axis=1, keepdims=True)
  out_ref[...] = y / jnp.maximum(m, 1e-8)


_normalize = pl.pallas_call(
    _norm_body,
    out_shape=jax.ShapeDtypeStruct((B, ROWS), jnp.float32),
)


@jax.jit
def kernel(p0, A_vals, A_rows, A_cols):
  p_img = p0[:, 0, :, :]
  p_vec = jnp.transpose(p_img, (0, 2, 1)).reshape(B, COLS)
  bits = jax.lax.bitcast_convert_type(
      p_vec.astype(jnp.bfloat16), jnp.uint16).astype(jnp.uint32)
  p01 = jax.lax.bitcast_convert_type((bits[0] << 16) | bits[1], jnp.int32)
  p23 = jax.lax.bitcast_convert_type((bits[2] << 16) | bits[3], jnp.int32)
  ypart = _sc_spmv(A_vals, A_rows, A_cols, p01, p23)
  ynorm = _normalize(ypart)
  return ynorm.reshape(B, 1, M, L)


# speedup vs baseline: 1.0154x; 1.0154x over previous
import functools

import jax
import jax.numpy as jnp
from jax import lax
from jax.experimental import pallas as pl
from jax.experimental.pallas import tpu as pltpu
from jax.experimental.pallas import tpu_sc as plsc

M, L, H, W = 128, 1024, 256, 256
NNZ = 8_000_000
B = 4
ROWS = M * L
COLS = H * W

WSZ = 4000
NWIN = NNZ // WSZ
NCORE = 2
NSUB = 16
WIN_PER_CORE = NWIN // NCORE
SUB_Q = WIN_PER_CORE // NSUB
SUB_R = WIN_PER_CORE - SUB_Q * NSUB

PSLICE = COLS // NSUB
YSLICE = ROWS // NSUB
ZB = 2048


def _sc_body(vals_hbm, rows_hbm, cols_hbm, p01_hbm, p23_hbm, out_hbm,
             p01_sh, p23_sh,
             y0_sh, y1_sh, y2_sh, y3_sh,
             vals_a, rows_a, cols_a, ppa01, ppa23, ua0, ua1, ua2, ua3,
             vals_b, rows_b, cols_b, ppb01, ppb23, ub0, ub1, ub2, ub3,
             zbuf, sem_lin, sem_gat, sem_sca):
  c = lax.axis_index("c")
  s = lax.axis_index("s")
  y_shs = [y0_sh, y1_sh, y2_sh, y3_sh]

  def _zero_zbuf(j, _):
    zbuf[pl.ds(j * 16, 16)] = jnp.zeros((16,), jnp.float32)
    return _
  lax.fori_loop(0, ZB // 16, _zero_zbuf, None)

  for b in range(B):
    for k in range(YSLICE // ZB):
      pltpu.sync_copy(zbuf, y_shs[b].at[pl.ds(s * YSLICE + k * ZB, ZB)])
  pltpu.sync_copy(p01_hbm.at[pl.ds(s * PSLICE, PSLICE)],
                  p01_sh.at[pl.ds(s * PSLICE, PSLICE)])
  pltpu.sync_copy(p23_hbm.at[pl.ds(s * PSLICE, PSLICE)],
                  p23_sh.at[pl.ds(s * PSLICE, PSLICE)])
  plsc.subcore_barrier()

  start = c * WIN_PER_CORE + s * SUB_Q + jnp.minimum(s, SUB_R)
  nwin = SUB_Q + jnp.where(s < SUB_R, 1, 0)
  SETS = (
      (vals_a, rows_a, cols_a, (ppa01, ppa23), (ua0, ua1, ua2, ua3)),
      (vals_b, rows_b, cols_b, (ppb01, ppb23), (ub0, ub1, ub2, ub3)),
  )

  def _issue_linear(w, t):
    vals_v, rows_v, cols_v, _, _ = SETS[t]
    base = (start + w) * WSZ
    pltpu.async_copy(vals_hbm.at[pl.ds(base, WSZ)], vals_v, sem_lin)
    pltpu.async_copy(rows_hbm.at[pl.ds(base, WSZ)], rows_v, sem_lin)
    pltpu.async_copy(cols_hbm.at[pl.ds(base, WSZ)], cols_v, sem_lin)

  def _drain_linear(t):
    vals_v, rows_v, cols_v, _, _ = SETS[t]
    pltpu.make_async_copy(vals_hbm.at[pl.ds(0, WSZ)], vals_v, sem_lin).wait()
    pltpu.make_async_copy(rows_hbm.at[pl.ds(0, WSZ)], rows_v, sem_lin).wait()
    pltpu.make_async_copy(cols_hbm.at[pl.ds(0, WSZ)], cols_v, sem_lin).wait()

  def _issue_gathers(t):
    _, _, cols_v, pps, _ = SETS[t]
    pltpu.async_copy(p01_sh.at[cols_v], pps[0], sem_gat)
    pltpu.async_copy(p23_sh.at[cols_v], pps[1], sem_gat)

  def _drain_gathers(t):
    _, _, cols_v, pps, _ = SETS[t]
    pltpu.make_async_copy(p01_sh.at[cols_v], pps[0], sem_gat).wait()
    pltpu.make_async_copy(p23_sh.at[cols_v], pps[1], sem_gat).wait()

  def _issue_scatters(t):
    _, rows_v, _, _, uds = SETS[t]
    for b in range(B):
      pltpu.async_copy(uds[b], y_shs[b].at[rows_v], sem_sca, add=True)

  def _drain_scatters(t):
    _, rows_v, _, _, uds = SETS[t]
    for b in range(B):
      pltpu.make_async_copy(uds[b], y_shs[b].at[rows_v], sem_sca).wait()

  def _mul(t):
    vals_v, _, _, pps, uds = SETS[t]
    mask_hi = jnp.int32(-65536)

    def body(j, _):
      sl = pl.ds(j * 16, 16)
      v = vals_v[sl]
      w01 = pps[0][sl]
      w23 = pps[1][sl]
      bc = jax.lax.bitcast_convert_type
      uds[0][sl] = bc(w01 & mask_hi, jnp.float32) * v
      uds[1][sl] = bc(w01 << 16, jnp.float32) * v
      uds[2][sl] = bc(w23 & mask_hi, jnp.float32) * v
      uds[3][sl] = bc(w23 << 16, jnp.float32) * v
      return _
    lax.fori_loop(0, WSZ // 16, body, None)

  npair = nwin // 2
  tail = nwin - 2 * npair

  _issue_linear(0, 0)
  _drain_linear(0)
  _issue_gathers(0)

  def _pair(q, _):
    wb = 2 * q + 1
    _drain_gathers(0)

    @pl.when(q >= 1)
    def _():
      _drain_scatters(1)
    _issue_linear(wb, 1)
    _mul(0)
    _issue_scatters(0)
    _drain_linear(1)
    _issue_gathers(1)
    _drain_gathers(1)
    _drain_scatters(0)

    @pl.when(wb + 1 < nwin)
    def _():
      _issue_linear(wb + 1, 0)
    _mul(1)
    _issue_scatters(1)

    @pl.when(wb + 1 < nwin)
    def _():
      _drain_linear(0)
      _issue_gathers(0)
    return _

  lax.fori_loop(0, npair, _pair, None)

  @pl.when(tail == 1)
  def _():
    _drain_gathers(0)
    _drain_scatters(1)
    _mul(0)
    _issue_scatters(0)
    _drain_scatters(0)

  @pl.when(tail == 0)
  def _():
    _drain_scatters(1)
  plsc.subcore_barrier()

  for b in range(B):
    pltpu.sync_copy(y_shs[b].at[pl.ds(s * YSLICE, YSLICE)],
                    out_hbm.at[c, b, pl.ds(s * YSLICE, YSLICE)])


_sc_spmv = functools.partial(
    pl.kernel,
    out_type=jax.ShapeDtypeStruct((NCORE, B, ROWS), jnp.float32),
    mesh=plsc.VectorSubcoreMesh(core_axis_name="c", subcore_axis_name="s"),
    scratch_types=[
        pltpu.VMEM_SHARED((COLS,), jnp.int32),
        pltpu.VMEM_SHARED((COLS,), jnp.int32),
        pltpu.VMEM_SHARED((ROWS,), jnp.float32),
        pltpu.VMEM_SHARED((ROWS,), jnp.float32),
        pltpu.VMEM_SHARED((ROWS,), jnp.float32),
        pltpu.VMEM_SHARED((ROWS,), jnp.float32),
        pltpu.VMEM((WSZ,), jnp.float32),
        pltpu.VMEM((WSZ,), jnp.int32),
        pltpu.VMEM((WSZ,), jnp.int32),
        pltpu.VMEM((WSZ,), jnp.int32),
        pltpu.VMEM((WSZ,), jnp.int32),
        pltpu.VMEM((WSZ,), jnp.float32),
        pltpu.VMEM((WSZ,), jnp.float32),
        pltpu.VMEM((WSZ,), jnp.float32),
        pltpu.VMEM((WSZ,), jnp.float32),
        pltpu.VMEM((WSZ,), jnp.float32),
        pltpu.VMEM((WSZ,), jnp.int32),
        pltpu.VMEM((WSZ,), jnp.int32),
        pltpu.VMEM((WSZ,), jnp.int32),
        pltpu.VMEM((WSZ,), jnp.int32),
        pltpu.VMEM((WSZ,), jnp.float32),
        pltpu.VMEM((WSZ,), jnp.float32),
        pltpu.VMEM((WSZ,), jnp.float32),
        pltpu.VMEM((WSZ,), jnp.float32),
        pltpu.VMEM((ZB,), jnp.float32),
        pltpu.SemaphoreType.DMA,
        pltpu.SemaphoreType.DMA,
        pltpu.SemaphoreType.DMA,
    ],
)(_sc_body)


def _norm_body(ypart_ref, out_ref):
  y = ypart_ref[0] + ypart_ref[1]
  m = jnp.max(y, axis=1, keepdims=True)
  out_ref[...] = y / jnp.maximum(m, 1e-8)


_normalize = pl.pallas_call(
    _norm_body,
    out_shape=jax.ShapeDtypeStruct((B, ROWS), jnp.float32),
)


@jax.jit
def kernel(p0, A_vals, A_rows, A_cols):
  p_img = p0[:, 0, :, :]
  p_vec = jnp.transpose(p_img, (0, 2, 1)).reshape(B, COLS)
  bits = jax.lax.bitcast_convert_type(
      p_vec.astype(jnp.bfloat16), jnp.uint16).astype(jnp.uint32)
  p01 = jax.lax.bitcast_convert_type((bits[0] << 16) | bits[1], jnp.int32)
  p23 = jax.lax.bitcast_convert_type((bits[2] << 16) | bits[3], jnp.int32)
  ypart = _sc_spmv(A_vals, A_rows, A_cols, p01, p23)
  ynorm = _normalize(ypart)
  return ynorm.reshape(B, 1, M, L)


# speedup vs baseline: 1.0312x; 1.0156x over previous
import functools

import jax
import jax.numpy as jnp
from jax import lax
from jax.experimental import pallas as pl
from jax.experimental.pallas import tpu as pltpu
from jax.experimental.pallas import tpu_sc as plsc

M, L, H, W = 128, 1024, 256, 256
NNZ = 8_000_000
B = 4
ROWS = M * L
COLS = H * W

WSZ = 4000
NWIN = NNZ // WSZ
NCORE = 2
NSUB = 16
WIN_PER_CORE = NWIN // NCORE
SUB_Q = WIN_PER_CORE // NSUB
SUB_R = WIN_PER_CORE - SUB_Q * NSUB

PSLICE = COLS // NSUB
YSLICE = ROWS // NSUB
ZB = 2048


def _sc_body(vals_hbm, rows_hbm, cols_hbm, p01_hbm, p23_hbm, out_hbm,
             p01_sh, p23_sh,
             y0_sh, y1_sh, y2_sh, y3_sh,
             vals_a, rows_a, cols_a, ppa01, ppa23, ua0, ua1, ua2, ua3,
             vals_b, rows_b, cols_b, ppb01, ppb23, ub0, ub1, ub2, ub3,
             zbuf, sem_lin, sem_gat, sem_sca):
  c = lax.axis_index("c")
  s = lax.axis_index("s")
  y_shs = [y0_sh, y1_sh, y2_sh, y3_sh]

  def _zero_zbuf(j, _):
    zbuf[pl.ds(j * 16, 16)] = jnp.zeros((16,), jnp.float32)
    return _
  lax.fori_loop(0, ZB // 16, _zero_zbuf, None)

  for b in range(B):
    for k in range(YSLICE // ZB):
      pltpu.sync_copy(zbuf, y_shs[b].at[pl.ds(s * YSLICE + k * ZB, ZB)])
  pltpu.sync_copy(p01_hbm.at[pl.ds(s * PSLICE, PSLICE)],
                  p01_sh.at[pl.ds(s * PSLICE, PSLICE)])
  pltpu.sync_copy(p23_hbm.at[pl.ds(s * PSLICE, PSLICE)],
                  p23_sh.at[pl.ds(s * PSLICE, PSLICE)])
  plsc.subcore_barrier()

  start = c * WIN_PER_CORE + s * SUB_Q + jnp.minimum(s, SUB_R)
  nwin = SUB_Q + jnp.where(s < SUB_R, 1, 0)
  SETS = (
      (vals_a, rows_a, cols_a, (ppa01, ppa23), (ua0, ua1, ua2, ua3)),
      (vals_b, rows_b, cols_b, (ppb01, ppb23), (ub0, ub1, ub2, ub3)),
  )

  def _issue_linear(w, t):
    vals_v, rows_v, cols_v, _, _ = SETS[t]
    base = (start + w) * WSZ
    pltpu.async_copy(vals_hbm.at[pl.ds(base, WSZ)], vals_v, sem_lin)
    pltpu.async_copy(rows_hbm.at[pl.ds(base, WSZ)], rows_v, sem_lin)
    pltpu.async_copy(cols_hbm.at[pl.ds(base, WSZ)], cols_v, sem_lin)

  def _drain_linear(t):
    vals_v, rows_v, cols_v, _, _ = SETS[t]
    pltpu.make_async_copy(vals_hbm.at[pl.ds(0, WSZ)], vals_v, sem_lin).wait()
    pltpu.make_async_copy(rows_hbm.at[pl.ds(0, WSZ)], rows_v, sem_lin).wait()
    pltpu.make_async_copy(cols_hbm.at[pl.ds(0, WSZ)], cols_v, sem_lin).wait()

  def _issue_gathers(t):
    _, _, cols_v, pps, _ = SETS[t]
    pltpu.async_copy(p01_sh.at[cols_v], pps[0], sem_gat)
    pltpu.async_copy(p23_sh.at[cols_v], pps[1], sem_gat)

  def _drain_gathers(t):
    _, _, cols_v, pps, _ = SETS[t]
    pltpu.make_async_copy(p01_sh.at[cols_v], pps[0], sem_gat).wait()
    pltpu.make_async_copy(p23_sh.at[cols_v], pps[1], sem_gat).wait()

  def _issue_scatters(t):
    _, rows_v, _, _, uds = SETS[t]
    for b in range(B):
      pltpu.async_copy(uds[b], y_shs[b].at[rows_v], sem_sca, add=True)

  def _drain_scatters(t):
    _, rows_v, _, _, uds = SETS[t]
    for b in range(B):
      pltpu.make_async_copy(uds[b], y_shs[b].at[rows_v], sem_sca).wait()

  def _mul(t):
    vals_v, _, _, pps, uds = SETS[t]
    mask_hi = jnp.int32(-65536)

    @plsc.parallel_loop(0, WSZ // 16, unroll=4)
    def _(j):
      sl = pl.ds(j * 16, 16)
      v = vals_v[sl]
      w01 = pps[0][sl]
      w23 = pps[1][sl]
      bc = jax.lax.bitcast_convert_type
      uds[0][sl] = bc(w01 & mask_hi, jnp.float32) * v
      uds[1][sl] = bc(w01 << 16, jnp.float32) * v
      uds[2][sl] = bc(w23 & mask_hi, jnp.float32) * v
      uds[3][sl] = bc(w23 << 16, jnp.float32) * v

  npair = nwin // 2
  tail = nwin - 2 * npair

  _issue_linear(0, 0)
  _drain_linear(0)
  _issue_gathers(0)

  def _pair(q, _):
    wb = 2 * q + 1
    _drain_gathers(0)

    @pl.when(q >= 1)
    def _():
      _drain_scatters(1)
    _issue_linear(wb, 1)
    _mul(0)
    _issue_scatters(0)
    _drain_linear(1)
    _issue_gathers(1)
    _drain_gathers(1)
    _drain_scatters(0)

    @pl.when(wb + 1 < nwin)
    def _():
      _issue_linear(wb + 1, 0)
    _mul(1)
    _issue_scatters(1)

    @pl.when(wb + 1 < nwin)
    def _():
      _drain_linear(0)
      _issue_gathers(0)
    return _

  lax.fori_loop(0, npair, _pair, None)

  @pl.when(tail == 1)
  def _():
    _drain_gathers(0)
    _drain_scatters(1)
    _mul(0)
    _issue_scatters(0)
    _drain_scatters(0)

  @pl.when(tail == 0)
  def _():
    _drain_scatters(1)
  plsc.subcore_barrier()

  for b in range(B):
    pltpu.sync_copy(y_shs[b].at[pl.ds(s * YSLICE, YSLICE)],
                    out_hbm.at[c, b, pl.ds(s * YSLICE, YSLICE)])


_sc_spmv = functools.partial(
    pl.kernel,
    out_type=jax.ShapeDtypeStruct((NCORE, B, ROWS), jnp.float32),
    mesh=plsc.VectorSubcoreMesh(core_axis_name="c", subcore_axis_name="s"),
    scratch_types=[
        pltpu.VMEM_SHARED((COLS,), jnp.int32),
        pltpu.VMEM_SHARED((COLS,), jnp.int32),
        pltpu.VMEM_SHARED((ROWS,), jnp.float32),
        pltpu.VMEM_SHARED((ROWS,), jnp.float32),
        pltpu.VMEM_SHARED((ROWS,), jnp.float32),
        pltpu.VMEM_SHARED((ROWS,), jnp.float32),
        pltpu.VMEM((WSZ,), jnp.float32),
        pltpu.VMEM((WSZ,), jnp.int32),
        pltpu.VMEM((WSZ,), jnp.int32),
        pltpu.VMEM((WSZ,), jnp.int32),
        pltpu.VMEM((WSZ,), jnp.int32),
        pltpu.VMEM((WSZ,), jnp.float32),
        pltpu.VMEM((WSZ,), jnp.float32),
        pltpu.VMEM((WSZ,), jnp.float32),
        pltpu.VMEM((WSZ,), jnp.float32),
        pltpu.VMEM((WSZ,), jnp.float32),
        pltpu.VMEM((WSZ,), jnp.int32),
        pltpu.VMEM((WSZ,), jnp.int32),
        pltpu.VMEM((WSZ,), jnp.int32),
        pltpu.VMEM((WSZ,), jnp.int32),
        pltpu.VMEM((WSZ,), jnp.float32),
        pltpu.VMEM((WSZ,), jnp.float32),
        pltpu.VMEM((WSZ,), jnp.float32),
        pltpu.VMEM((WSZ,), jnp.float32),
        pltpu.VMEM((ZB,), jnp.float32),
        pltpu.SemaphoreType.DMA,
        pltpu.SemaphoreType.DMA,
        pltpu.SemaphoreType.DMA,
    ],
)(_sc_body)


def _norm_body(ypart_ref, out_ref):
  y = ypart_ref[0] + ypart_ref[1]
  m = jnp.max(y, axis=1, keepdims=True)
  out_ref[...] = y / jnp.maximum(m, 1e-8)


_normalize = pl.pallas_call(
    _norm_body,
    out_shape=jax.ShapeDtypeStruct((B, ROWS), jnp.float32),
)


@jax.jit
def kernel(p0, A_vals, A_rows, A_cols):
  p_img = p0[:, 0, :, :]
  p_vec = jnp.transpose(p_img, (0, 2, 1)).reshape(B, COLS)
  bits = jax.lax.bitcast_convert_type(
      p_vec.astype(jnp.bfloat16), jnp.uint16).astype(jnp.uint32)
  p01 = jax.lax.bitcast_convert_type((bits[0] << 16) | bits[1], jnp.int32)
  p23 = jax.lax.bitcast_convert_type((bits[2] << 16) | bits[3], jnp.int32)
  ypart = _sc_spmv(A_vals, A_rows, A_cols, p01, p23)
  ynorm = _normalize(ypart)
  return ynorm.reshape(B, 1, M, L)
